# baseline (speedup 1.0000x reference)
import functools

import jax
import jax.numpy as jnp
from jax import lax
from jax.experimental import pallas as pl
from jax.experimental.pallas import tpu as pltpu
from jax.experimental.pallas import tpu_sc as plsc

E = 8
N = 8192
D = 1024

NH = N // 2
NC = 2
NS = 16
NW = NC * NS
CPW = NH // NW
LANES = 16
NCH = CPW // LANES

T = 512
NUM_TILES = NH // T
MAX_ITEMS = NUM_TILES + E


def _wid():
    return lax.axis_index("s") * NC + lax.axis_index("c")


@functools.cache
def _mesh():
    return plsc.VectorSubcoreMesh(core_axis_name="c", subcore_axis_name="s")


def _sc_params():
    return pltpu.CompilerParams(needs_layout_passes=False)


@functools.cache
def _sc_count(base0):
    def body(assign_hbm, cnts_hbm, asg_v, cnt_v):
        w = _wid()
        base = base0 + w * CPW
        pltpu.sync_copy(assign_hbm.at[pl.ds(base, CPW)], asg_v)
        lane = lax.iota(jnp.int32, LANES)
        cnt = jnp.zeros((LANES,), jnp.int32)
        for j in range(NCH):
            vals = asg_v[pl.ds(j * LANES, LANES)]
            for e in range(E):
                pc = jnp.sum((vals == e).astype(jnp.int32))
                cnt = cnt + jnp.where(lane == e, pc, 0)
        cnt_v[...] = cnt
        pltpu.sync_copy(cnt_v, cnts_hbm.at[w])

    return pl.kernel(
        body,
        out_type=jax.ShapeDtypeStruct((NW, LANES), jnp.int32),
        mesh=_mesh(),
        scratch_types=[
            pltpu.VMEM((CPW,), jnp.int32),
            pltpu.VMEM((LANES,), jnp.int32),
        ],
        compiler_params=_sc_params(),
    )


@functools.cache
def _sc_dispatch(base0):
    def body(x_hbm, assign_hbm, cnts_hbm, xs_hbm, dest_hbm,
             asg_v, cnt_v, dest_v, xbuf_v,
             ls0, ls1, ls2, ls3, ls4, ls5,
             ss0, ss1, ss2, ss3, ss4, ss5):
        w = _wid()
        base = base0 + w * CPW
        lsem = (ls0, ls1, ls2, ls3, ls4, ls5)
        ssem = (ss0, ss1, ss2, ss3, ss4, ss5)

        h_ld = {}
        for j in range(3):
            h_ld[j] = pltpu.async_copy(
                x_hbm.at[pl.ds(base + j * LANES, LANES)], xbuf_v.at[j],
                lsem[j])

        pltpu.sync_copy(assign_hbm.at[pl.ds(base, CPW)], asg_v)
        pltpu.sync_copy(cnts_hbm, cnt_v)

        lane = lax.iota(jnp.int32, LANES)
        tot = jnp.zeros((LANES,), jnp.int32)
        pre = jnp.zeros((LANES,), jnp.int32)
        for wp in range(NW):
            row = cnt_v[wp]
            tot = tot + row
            pred = jnp.full((LANES,), wp, jnp.int32) < w
            pre = pre + jnp.where(pred, row, 0)
        off = plsc.cumsum(tot) - tot
        cur = off + pre

        for j in range(NCH):
            vals = asg_v[pl.ds(j * LANES, LANES)]
            dest = jnp.zeros((LANES,), jnp.int32)
            for e in range(E):
                m = vals == e
                mi = m.astype(jnp.int32)
                cs = plsc.cumsum(mi)
                cur_e = jnp.sum(jnp.where(lane == e, cur, 0))
                tot_e = jnp.sum(mi)
                dest = jnp.where(m, cur_e + cs - 1, dest)
                cur = cur + jnp.where(lane == e, tot_e, 0)
            dest_v[j] = dest

        pltpu.sync_copy(dest_v, dest_hbm.at[w])

        h_sc = {}
        for j in range(NCH):
            bb = j % 6
            h_ld[j].wait()
            dv = dest_v[j]
            h_sc[j] = pltpu.async_copy(xbuf_v.at[bb], xs_hbm.at[dv],
                                       ssem[bb])
            if j + 3 < NCH:
                if j - 3 >= 0:
                    h_sc[j - 3].wait()
                h_ld[j + 3] = pltpu.async_copy(
                    x_hbm.at[pl.ds(base + (j + 3) * LANES, LANES)],
                    xbuf_v.at[(j + 3) % 6], lsem[(j + 3) % 6])
        for j in range(max(0, NCH - 6), NCH):
            h_sc[j].wait()

    return pl.kernel(
        body,
        out_type=(
            jax.ShapeDtypeStruct((NH, D), jnp.float32),
            jax.ShapeDtypeStruct((NW, NCH, LANES), jnp.int32),
        ),
        mesh=_mesh(),
        scratch_types=[
            pltpu.VMEM((CPW,), jnp.int32),
            pltpu.VMEM((NW, LANES), jnp.int32),
            pltpu.VMEM((NCH, LANES), jnp.int32),
            pltpu.VMEM((6, LANES, D), jnp.float32),
        ] + [pltpu.SemaphoreType.DMA] * 12,
        compiler_params=_sc_params(),
    )


def _sc_gather_body(ysa_hbm, ysb_hbm, desta_hbm, destb_hbm, out_hbm,
                    dav, dbv, ybuf_v,
                    g0, g1, g2, g3, g4, g5, s0, s1, s2, s3, s4, s5):
    w = _wid()
    gsem = (g0, g1, g2, g3, g4, g5)
    ssem = (s0, s1, s2, s3, s4, s5)
    pltpu.sync_copy(desta_hbm.at[w], dav)
    pltpu.sync_copy(destb_hbm.at[w], dbv)

    def item(j):
        h, jj = divmod(j, NCH)
        src = ysa_hbm if h == 0 else ysb_hbm
        drow = (dav if h == 0 else dbv).at[jj]
        obase = h * NH + w * CPW + jj * LANES
        return src, drow, obase

    nit = 2 * NCH
    h_g, h_s = {}, {}
    for j in range(3):
        src, drow, _ = item(j)
        h_g[j] = pltpu.async_copy(src.at[drow], ybuf_v.at[j], gsem[j])
    for j in range(nit):
        bb = j % 6
        _, _, obase = item(j)
        h_g[j].wait()
        h_s[j] = pltpu.async_copy(
            ybuf_v.at[bb], out_hbm.at[pl.ds(obase, LANES)], ssem[bb])
        if j + 3 < nit:
            if j - 3 >= 0:
                h_s[j - 3].wait()
            src, drow, _ = item(j + 3)
            h_g[j + 3] = pltpu.async_copy(
                src.at[drow], ybuf_v.at[(j + 3) % 6], gsem[(j + 3) % 6])
    for j in range(nit - 6, nit):
        h_s[j].wait()


@functools.cache
def _sc_gather():
    return pl.kernel(
        _sc_gather_body,
        out_type=jax.ShapeDtypeStruct((N, D), jnp.float32),
        mesh=_mesh(),
        scratch_types=[
            pltpu.VMEM((NCH, LANES), jnp.int32),
            pltpu.VMEM((NCH, LANES), jnp.int32),
            pltpu.VMEM((6, LANES, D), jnp.float32),
        ] + [pltpu.SemaphoreType.DMA] * 12,
        compiler_params=_sc_params(),
    )


def _gmm_body(it_tile, it_e, it_start, it_end, it_valid,
              x_ref, w_ref, b_ref, out_ref, wb_ref):
    i = pl.program_id(0)
    start = it_start[i]
    end = it_end[i]
    tile = it_tile[i]

    prev_e = it_e[jnp.maximum(i - 1, 0)]

    @pl.when((i == 0) | (it_e[i] != prev_e))
    def _():
        wb_ref[...] = w_ref[0].astype(jnp.bfloat16)

    full = (start <= tile * T) & (end >= (tile + 1) * T)

    @pl.when((it_valid[i] == 1) & full)
    def _():
        out_ref[...] = jnp.dot(x_ref[...].astype(jnp.bfloat16), wb_ref[...],
                               preferred_element_type=jnp.float32) + b_ref[0]

    @pl.when((it_valid[i] == 1) & jnp.logical_not(full))
    def _():
        rows = tile * T + lax.broadcasted_iota(jnp.int32, (T, 1), 0)
        m = (rows >= start) & (rows < end)
        y = jnp.dot(x_ref[...].astype(jnp.bfloat16), wb_ref[...],
                    preferred_element_type=jnp.float32) + b_ref[0]
        out_ref[...] = jnp.where(m, y, out_ref[...])


def _gmm(it_tile, it_e, it_start, it_end, it_valid, xs, W, b):
    grid_spec = pltpu.PrefetchScalarGridSpec(
        num_scalar_prefetch=5,
        grid=(MAX_ITEMS,),
        in_specs=[
            pl.BlockSpec((T, D), lambda i, tl, ex, st, en, va: (tl[i], 0)),
            pl.BlockSpec((1, D, D), lambda i, tl, ex, st, en, va: (ex[i], 0, 0)),
            pl.BlockSpec((1, 1, D), lambda i, tl, ex, st, en, va: (ex[i], 0, 0)),
        ],
        out_specs=pl.BlockSpec((T, D), lambda i, tl, ex, st, en, va: (tl[i], 0)),
        scratch_shapes=[pltpu.VMEM((D, D), jnp.bfloat16)],
    )
    return pl.pallas_call(
        _gmm_body,
        grid_spec=grid_spec,
        out_shape=jax.ShapeDtypeStruct((NH, D), jnp.float32),
    )(it_tile, it_e, it_start, it_end, it_valid, xs, W,
      b.reshape(E, 1, D))


def _metadata(cnts):
    totals = jnp.sum(cnts, axis=0)[:E]
    ends = jnp.cumsum(totals)
    starts = ends - totals
    first_tile = starts // T
    last_tile_ex = (ends + T - 1) // T
    ntiles = jnp.where(totals > 0, last_tile_ex - first_tile, 0)
    csum = jnp.cumsum(ntiles)
    total_items = csum[-1]
    i = jnp.arange(MAX_ITEMS, dtype=jnp.int32)
    ic = jnp.minimum(i, total_items - 1)
    e_of = jnp.sum((ic[:, None] >= csum[None, :]).astype(jnp.int32), axis=1)
    onehot = e_of[:, None] == jnp.arange(E, dtype=jnp.int32)[None, :]

    def pick(tbl):
        return jnp.sum(jnp.where(onehot, tbl[None, :], 0), axis=1).astype(jnp.int32)

    it_tile = pick(first_tile) + (ic - (pick(csum) - pick(ntiles)))
    it_e = e_of.astype(jnp.int32)
    it_valid = (i < total_items).astype(jnp.int32)
    return it_tile, it_e, pick(starts), pick(ends), it_valid


def kernel(x, W, b, assign):
    assign = assign.astype(jnp.int32)
    cnts_a = _sc_count(0)(assign)
    cnts_b = _sc_count(NH)(assign)
    xs_a, dest_a = _sc_dispatch(0)(x, assign, cnts_a)
    xs_b, dest_b = _sc_dispatch(NH)(x, assign, cnts_b)
    ys_a = _gmm(*_metadata(cnts_a), xs_a, W, b)
    ys_b = _gmm(*_metadata(cnts_b), xs_b, W, b)
    return _sc_gather()(ys_a, ys_b, dest_a, dest_b)


# speedup vs baseline: 1.0745x; 1.0745x over previous
import functools

import jax
import jax.numpy as jnp
from jax import lax
from jax.experimental import pallas as pl
from jax.experimental.pallas import tpu as pltpu
from jax.experimental.pallas import tpu_sc as plsc

E = 8
N = 8192
D = 1024

NC = 2
NS = 16
NW = NC * NS
CPW = N // NW
LANES = 16

T = 256
NUM_TILES = N // T
MAX_ITEMS = NUM_TILES + E

def _wid():
    return lax.axis_index("s") * NC + lax.axis_index("c")


@functools.cache
def _mesh():
    return plsc.VectorSubcoreMesh(core_axis_name="c", subcore_axis_name="s")


def _sc_params():
    return pltpu.CompilerParams(needs_layout_passes=False)


def _sc_count_body(assign_hbm, cnts_hbm, asg_v, cnt_v):
    w = _wid()
    base = w * CPW
    pltpu.sync_copy(assign_hbm.at[pl.ds(base, CPW)], asg_v)
    lane = lax.iota(jnp.int32, LANES)
    cnt = jnp.zeros((LANES,), jnp.int32)
    for j in range(CPW // LANES):
        vals = asg_v[pl.ds(j * LANES, LANES)]
        for e in range(E):
            pc = jnp.sum((vals == e).astype(jnp.int32))
            cnt = cnt + jnp.where(lane == e, pc, 0)
    cnt_v[...] = cnt
    pltpu.sync_copy(cnt_v, cnts_hbm.at[w])


@functools.cache
def _sc_count():
    return pl.kernel(
        _sc_count_body,
        out_type=jax.ShapeDtypeStruct((NW, LANES), jnp.int32),
        mesh=_mesh(),
        scratch_types=[
            pltpu.VMEM((CPW,), jnp.int32),
            pltpu.VMEM((LANES,), jnp.int32),
        ],
        compiler_params=_sc_params(),
    )


_NCHUNK = CPW // LANES


def _sc_dispatch_body(x_hbm, assign_hbm, cnts_hbm, xs_hbm, dest_hbm,
                      asg_v, cnt_v, dest_v, xbuf_v,
                      ls0, ls1, ls2, ls3, ls4, ls5,
                      ss0, ss1, ss2, ss3, ss4, ss5):
    w = _wid()
    base = w * CPW
    lsem = (ls0, ls1, ls2, ls3, ls4, ls5)
    ssem = (ss0, ss1, ss2, ss3, ss4, ss5)

    h_ld = {}
    for j in range(3):
        h_ld[j] = pltpu.async_copy(
            x_hbm.at[pl.ds(base + j * LANES, LANES)], xbuf_v.at[j], lsem[j])

    pltpu.sync_copy(assign_hbm.at[pl.ds(base, CPW)], asg_v)
    pltpu.sync_copy(cnts_hbm, cnt_v)

    lane = lax.iota(jnp.int32, LANES)
    tot = jnp.zeros((LANES,), jnp.int32)
    pre = jnp.zeros((LANES,), jnp.int32)
    for wp in range(NW):
        row = cnt_v[wp]
        tot = tot + row
        pred = jnp.full((LANES,), wp, jnp.int32) < w
        pre = pre + jnp.where(pred, row, 0)
    off = plsc.cumsum(tot) - tot
    cur = off + pre

    for j in range(_NCHUNK):
        vals = asg_v[pl.ds(j * LANES, LANES)]
        dest = jnp.zeros((LANES,), jnp.int32)
        for e in range(E):
            m = vals == e
            mi = m.astype(jnp.int32)
            cs = plsc.cumsum(mi)
            cur_e = jnp.sum(jnp.where(lane == e, cur, 0))
            tot_e = jnp.sum(mi)
            dest = jnp.where(m, cur_e + cs - 1, dest)
            cur = cur + jnp.where(lane == e, tot_e, 0)
        dest_v[j] = dest

    pltpu.sync_copy(dest_v, dest_hbm.at[w])

    h_sc = {}
    for j in range(_NCHUNK):
        bb = j % 6
        h_ld[j].wait()
        dv = dest_v[j]
        h_sc[j] = pltpu.async_copy(xbuf_v.at[bb], xs_hbm.at[dv], ssem[bb])
        if j + 3 < _NCHUNK:
            if j - 3 >= 0:
                h_sc[j - 3].wait()
            h_ld[j + 3] = pltpu.async_copy(
                x_hbm.at[pl.ds(base + (j + 3) * LANES, LANES)],
                xbuf_v.at[(j + 3) % 6], lsem[(j + 3) % 6])
    for j in range(_NCHUNK - 6, _NCHUNK):
        h_sc[j].wait()


@functools.cache
def _sc_dispatch():
    return pl.kernel(
        _sc_dispatch_body,
        out_type=(
            jax.ShapeDtypeStruct((N, D), jnp.float32),
            jax.ShapeDtypeStruct((NW, _NCHUNK, LANES), jnp.int32),
        ),
        mesh=_mesh(),
        scratch_types=[
            pltpu.VMEM((CPW,), jnp.int32),
            pltpu.VMEM((NW, LANES), jnp.int32),
            pltpu.VMEM((_NCHUNK, LANES), jnp.int32),
            pltpu.VMEM((6, LANES, D), jnp.float32),
        ] + [pltpu.SemaphoreType.DMA] * 12,
        compiler_params=_sc_params(),
    )


def _sc_gather_body(ys_hbm, dest_hbm, out_hbm, dest_v, ybuf_v,
                    g0, g1, g2, g3, g4, g5, s0, s1, s2, s3, s4, s5):
    w = _wid()
    base = w * CPW
    gsem = (g0, g1, g2, g3, g4, g5)
    ssem = (s0, s1, s2, s3, s4, s5)
    pltpu.sync_copy(dest_hbm.at[w], dest_v)
    h_g, h_s = {}, {}
    for j in range(3):
        h_g[j] = pltpu.async_copy(
            ys_hbm.at[dest_v.at[j]], ybuf_v.at[j], gsem[j])
    for j in range(_NCHUNK):
        bb = j % 6
        h_g[j].wait()
        h_s[j] = pltpu.async_copy(
            ybuf_v.at[bb], out_hbm.at[pl.ds(base + j * LANES, LANES)],
            ssem[bb])
        if j + 3 < _NCHUNK:
            if j - 3 >= 0:
                h_s[j - 3].wait()
            h_g[j + 3] = pltpu.async_copy(
                ys_hbm.at[dest_v.at[j + 3]], ybuf_v.at[(j + 3) % 6],
                gsem[(j + 3) % 6])
    for j in range(_NCHUNK - 6, _NCHUNK):
        h_s[j].wait()


@functools.cache
def _sc_gather():
    return pl.kernel(
        _sc_gather_body,
        out_type=jax.ShapeDtypeStruct((N, D), jnp.float32),
        mesh=_mesh(),
        scratch_types=[
            pltpu.VMEM((_NCHUNK, LANES), jnp.int32),
            pltpu.VMEM((6, LANES, D), jnp.float32),
        ] + [pltpu.SemaphoreType.DMA] * 12,
        compiler_params=_sc_params(),
    )


def _gmm_body(it_tile, it_e, it_start, it_end, it_valid,
              x_ref, w_ref, b_ref, out_ref, wb_ref):
    i = pl.program_id(0)
    start = it_start[i]
    end = it_end[i]
    tile = it_tile[i]

    prev_e = it_e[jnp.maximum(i - 1, 0)]

    @pl.when((i == 0) | (it_e[i] != prev_e))
    def _():
        wb_ref[...] = w_ref[0].astype(jnp.bfloat16)

    full = (start <= tile * T) & (end >= (tile + 1) * T)

    @pl.when((it_valid[i] == 1) & full)
    def _():
        out_ref[...] = jnp.dot(x_ref[...].astype(jnp.bfloat16), wb_ref[...],
                               preferred_element_type=jnp.float32) + b_ref[0]

    @pl.when((it_valid[i] == 1) & jnp.logical_not(full))
    def _():
        rows = tile * T + lax.broadcasted_iota(jnp.int32, (T, 1), 0)
        m = (rows >= start) & (rows < end)
        y = jnp.dot(x_ref[...].astype(jnp.bfloat16), wb_ref[...],
                    preferred_element_type=jnp.float32) + b_ref[0]
        out_ref[...] = jnp.where(m, y, out_ref[...])


def _gmm(it_tile, it_e, it_start, it_end, it_valid, xs, W, b):
    grid_spec = pltpu.PrefetchScalarGridSpec(
        num_scalar_prefetch=5,
        grid=(MAX_ITEMS,),
        in_specs=[
            pl.BlockSpec((T, D), lambda i, tl, ex, st, en, va: (tl[i], 0)),
            pl.BlockSpec((1, D, D), lambda i, tl, ex, st, en, va: (ex[i], 0, 0)),
            pl.BlockSpec((1, 1, D), lambda i, tl, ex, st, en, va: (ex[i], 0, 0)),
        ],
        out_specs=pl.BlockSpec((T, D), lambda i, tl, ex, st, en, va: (tl[i], 0)),
        scratch_shapes=[pltpu.VMEM((D, D), jnp.bfloat16)],
    )
    return pl.pallas_call(
        _gmm_body,
        grid_spec=grid_spec,
        out_shape=jax.ShapeDtypeStruct((N, D), jnp.float32),
    )(it_tile, it_e, it_start, it_end, it_valid, xs, W,
      b.reshape(E, 1, D))


def kernel(x, W, b, assign):
    assign = assign.astype(jnp.int32)
    cnts = _sc_count()(assign)
    xs, dest = _sc_dispatch()(x, assign, cnts)

    totals = jnp.sum(cnts, axis=0)[:E]
    ends = jnp.cumsum(totals)
    starts = ends - totals
    first_tile = starts // T
    last_tile_ex = (ends + T - 1) // T
    ntiles = jnp.where(totals > 0, last_tile_ex - first_tile, 0)
    csum = jnp.cumsum(ntiles)
    total_items = csum[-1]
    i = jnp.arange(MAX_ITEMS, dtype=jnp.int32)
    ic = jnp.minimum(i, total_items - 1)
    e_of = jnp.sum((ic[:, None] >= csum[None, :]).astype(jnp.int32), axis=1)
    onehot = e_of[:, None] == jnp.arange(E, dtype=jnp.int32)[None, :]

    def pick(tbl):
        return jnp.sum(jnp.where(onehot, tbl[None, :], 0), axis=1).astype(jnp.int32)

    it_tile = pick(first_tile) + (ic - (pick(csum) - pick(ntiles)))
    it_e = e_of.astype(jnp.int32)
    it_start = pick(starts)
    it_end = pick(ends)
    it_valid = (i < total_items).astype(jnp.int32)

    ys = _gmm(it_tile, it_e, it_start, it_end, it_valid, xs, W, b)
    return _sc_gather()(ys, dest)


# speedup vs baseline: 1.1714x; 1.0901x over previous
import functools

import jax
import jax.numpy as jnp
from jax import lax
from jax.experimental import pallas as pl
from jax.experimental.pallas import tpu as pltpu
from jax.experimental.pallas import tpu_sc as plsc

E = 8
N = 8192
D = 1024

NC = 2
NS = 16
NW = NC * NS
CPW = N // NW
LANES = 16

T = 512
NUM_TILES = N // T
MAX_ITEMS = NUM_TILES + E

def _wid():
    return lax.axis_index("s") * NC + lax.axis_index("c")


@functools.cache
def _mesh():
    return plsc.VectorSubcoreMesh(core_axis_name="c", subcore_axis_name="s")


def _sc_params():
    return pltpu.CompilerParams(needs_layout_passes=False)


def _sc_count_body(assign_hbm, cnts_hbm, asg_v, cnt_v):
    w = _wid()
    base = w * CPW
    pltpu.sync_copy(assign_hbm.at[pl.ds(base, CPW)], asg_v)
    lane = lax.iota(jnp.int32, LANES)
    cnt = jnp.zeros((LANES,), jnp.int32)
    for j in range(CPW // LANES):
        vals = asg_v[pl.ds(j * LANES, LANES)]
        for e in range(E):
            pc = jnp.sum((vals == e).astype(jnp.int32))
            cnt = cnt + jnp.where(lane == e, pc, 0)
    cnt_v[...] = cnt
    pltpu.sync_copy(cnt_v, cnts_hbm.at[w])


@functools.cache
def _sc_count():
    return pl.kernel(
        _sc_count_body,
        out_type=jax.ShapeDtypeStruct((NW, LANES), jnp.int32),
        mesh=_mesh(),
        scratch_types=[
            pltpu.VMEM((CPW,), jnp.int32),
            pltpu.VMEM((LANES,), jnp.int32),
        ],
        compiler_params=_sc_params(),
    )


_NCHUNK = CPW // LANES


def _sc_dispatch_body(x_hbm, assign_hbm, cnts_hbm, xs_hbm, dest_hbm,
                      asg_v, cnt_v, dest_v, xbuf_v, lcnt_v,
                      ls0, ls1, ls2, ls3, ls4, ls5,
                      ss0, ss1, ss2, ss3, ss4, ss5):
    w = _wid()
    base = w * CPW
    lsem = (ls0, ls1, ls2, ls3, ls4, ls5)
    ssem = (ss0, ss1, ss2, ss3, ss4, ss5)

    h_ld = {}
    for j in range(3):
        h_ld[j] = pltpu.async_copy(
            x_hbm.at[pl.ds(base + j * LANES, LANES)], xbuf_v.at[j], lsem[j])

    pltpu.sync_copy(assign_hbm.at[pl.ds(base, CPW)], asg_v)
    pltpu.sync_copy(cnts_hbm, cnt_v)

    lane = lax.iota(jnp.int32, LANES)
    tot = jnp.zeros((LANES,), jnp.int32)
    pre = jnp.zeros((LANES,), jnp.int32)
    for wp in range(NW):
        row = cnt_v[wp]
        tot = tot + row
        pred = jnp.full((LANES,), wp, jnp.int32) < w
        pre = pre + jnp.where(pred, row, 0)
    off = plsc.cumsum(tot) - tot
    cur = off + pre

    ones = jnp.ones((LANES,), jnp.int32)
    for e in range(E):
        lcnt_v[e] = jnp.zeros((LANES,), jnp.int32)
    for j in range(_NCHUNK):
        vals = asg_v[pl.ds(j * LANES, LANES)]
        plsc.addupdate_scatter(lcnt_v, [vals, lane], ones)
    for e in range(E):
        row = lcnt_v[e]
        excl = plsc.cumsum(row) - row
        cur_e = jnp.sum(jnp.where(lane == e, cur, 0))
        lcnt_v[e] = excl + cur_e
    for j in range(_NCHUNK):
        vals = asg_v[pl.ds(j * LANES, LANES)]
        dest = plsc.load_gather(lcnt_v, [vals, lane])
        plsc.addupdate_scatter(lcnt_v, [vals, lane], ones)
        dest_v[j] = dest

    pltpu.sync_copy(dest_v, dest_hbm.at[w])

    h_sc = {}
    for j in range(_NCHUNK):
        bb = j % 6
        h_ld[j].wait()
        dv = dest_v[j]
        h_sc[j] = pltpu.async_copy(xbuf_v.at[bb], xs_hbm.at[dv], ssem[bb])
        if j + 3 < _NCHUNK:
            if j - 3 >= 0:
                h_sc[j - 3].wait()
            h_ld[j + 3] = pltpu.async_copy(
                x_hbm.at[pl.ds(base + (j + 3) * LANES, LANES)],
                xbuf_v.at[(j + 3) % 6], lsem[(j + 3) % 6])
    for j in range(_NCHUNK - 6, _NCHUNK):
        h_sc[j].wait()


@functools.cache
def _sc_dispatch():
    return pl.kernel(
        _sc_dispatch_body,
        out_type=(
            jax.ShapeDtypeStruct((N, D), jnp.float32),
            jax.ShapeDtypeStruct((NW, _NCHUNK, LANES), jnp.int32),
        ),
        mesh=_mesh(),
        scratch_types=[
            pltpu.VMEM((CPW,), jnp.int32),
            pltpu.VMEM((NW, LANES), jnp.int32),
            pltpu.VMEM((_NCHUNK, LANES), jnp.int32),
            pltpu.VMEM((6, LANES, D), jnp.float32),
            pltpu.VMEM((E, LANES), jnp.int32),
        ] + [pltpu.SemaphoreType.DMA] * 12,
        compiler_params=_sc_params(),
    )


def _sc_gather_body(ys_hbm, dest_hbm, out_hbm, dest_v, ybuf_v,
                    g0, g1, g2, g3, g4, g5, s0, s1, s2, s3, s4, s5):
    w = _wid()
    base = w * CPW
    gsem = (g0, g1, g2, g3, g4, g5)
    ssem = (s0, s1, s2, s3, s4, s5)
    pltpu.sync_copy(dest_hbm.at[w], dest_v)
    h_g, h_s = {}, {}
    for j in range(3):
        h_g[j] = pltpu.async_copy(
            ys_hbm.at[dest_v.at[j]], ybuf_v.at[j], gsem[j])
    for j in range(_NCHUNK):
        bb = j % 6
        h_g[j].wait()
        h_s[j] = pltpu.async_copy(
            ybuf_v.at[bb], out_hbm.at[pl.ds(base + j * LANES, LANES)],
            ssem[bb])
        if j + 3 < _NCHUNK:
            if j - 3 >= 0:
                h_s[j - 3].wait()
            h_g[j + 3] = pltpu.async_copy(
                ys_hbm.at[dest_v.at[j + 3]], ybuf_v.at[(j + 3) % 6],
                gsem[(j + 3) % 6])
    for j in range(_NCHUNK - 6, _NCHUNK):
        h_s[j].wait()


@functools.cache
def _sc_gather():
    return pl.kernel(
        _sc_gather_body,
        out_type=jax.ShapeDtypeStruct((N, D), jnp.float32),
        mesh=_mesh(),
        scratch_types=[
            pltpu.VMEM((_NCHUNK, LANES), jnp.int32),
            pltpu.VMEM((6, LANES, D), jnp.float32),
        ] + [pltpu.SemaphoreType.DMA] * 12,
        compiler_params=_sc_params(),
    )


def _gmm_body(it_tile, it_e, it_start, it_end, it_valid,
              x_ref, w_ref, b_ref, out_ref, wb_ref):
    i = pl.program_id(0)
    start = it_start[i]
    end = it_end[i]
    tile = it_tile[i]

    prev_e = it_e[jnp.maximum(i - 1, 0)]

    @pl.when((i == 0) | (it_e[i] != prev_e))
    def _():
        wb_ref[...] = w_ref[0].astype(jnp.bfloat16)

    full = (start <= tile * T) & (end >= (tile + 1) * T)

    @pl.when((it_valid[i] == 1) & full)
    def _():
        out_ref[...] = jnp.dot(x_ref[...].astype(jnp.bfloat16), wb_ref[...],
                               preferred_element_type=jnp.float32) + b_ref[0]

    @pl.when((it_valid[i] == 1) & jnp.logical_not(full))
    def _():
        rows = tile * T + lax.broadcasted_iota(jnp.int32, (T, 1), 0)
        m = (rows >= start) & (rows < end)
        y = jnp.dot(x_ref[...].astype(jnp.bfloat16), wb_ref[...],
                    preferred_element_type=jnp.float32) + b_ref[0]
        out_ref[...] = jnp.where(m, y, out_ref[...])


def _gmm(it_tile, it_e, it_start, it_end, it_valid, xs, W, b):
    grid_spec = pltpu.PrefetchScalarGridSpec(
        num_scalar_prefetch=5,
        grid=(MAX_ITEMS,),
        in_specs=[
            pl.BlockSpec((T, D), lambda i, tl, ex, st, en, va: (tl[i], 0)),
            pl.BlockSpec((1, D, D), lambda i, tl, ex, st, en, va: (ex[i], 0, 0)),
            pl.BlockSpec((1, 1, D), lambda i, tl, ex, st, en, va: (ex[i], 0, 0)),
        ],
        out_specs=pl.BlockSpec((T, D), lambda i, tl, ex, st, en, va: (tl[i], 0)),
        scratch_shapes=[pltpu.VMEM((D, D), jnp.bfloat16)],
    )
    return pl.pallas_call(
        _gmm_body,
        grid_spec=grid_spec,
        out_shape=jax.ShapeDtypeStruct((N, D), jnp.float32),
    )(it_tile, it_e, it_start, it_end, it_valid, xs, W,
      b.reshape(E, 1, D))


def kernel(x, W, b, assign):
    assign = assign.astype(jnp.int32)
    cnts = _sc_count()(assign)
    xs, dest = _sc_dispatch()(x, assign, cnts)

    totals = jnp.sum(cnts, axis=0)[:E]
    ends = jnp.cumsum(totals)
    starts = ends - totals
    first_tile = starts // T
    last_tile_ex = (ends + T - 1) // T
    ntiles = jnp.where(totals > 0, last_tile_ex - first_tile, 0)
    csum = jnp.cumsum(ntiles)
    total_items = csum[-1]
    i = jnp.arange(MAX_ITEMS, dtype=jnp.int32)
    ic = jnp.minimum(i, total_items - 1)
    e_of = jnp.sum((ic[:, None] >= csum[None, :]).astype(jnp.int32), axis=1)
    onehot = e_of[:, None] == jnp.arange(E, dtype=jnp.int32)[None, :]

    def pick(tbl):
        return jnp.sum(jnp.where(onehot, tbl[None, :], 0), axis=1).astype(jnp.int32)

    it_tile = pick(first_tile) + (ic - (pick(csum) - pick(ntiles)))
    it_e = e_of.astype(jnp.int32)
    it_start = pick(starts)
    it_end = pick(ends)
    it_valid = (i < total_items).astype(jnp.int32)

    ys = _gmm(it_tile, it_e, it_start, it_end, it_valid, xs, W, b)
    return _sc_gather()(ys, dest)


# speedup vs baseline: 1.1848x; 1.0114x over previous
import functools

import jax
import jax.numpy as jnp
from jax import lax
from jax.experimental import pallas as pl
from jax.experimental.pallas import tpu as pltpu
from jax.experimental.pallas import tpu_sc as plsc

E = 8
N = 8192
D = 1024

NC = 2
NS = 16
NW = NC * NS
CPW = N // NW
LANES = 16

T = 512
NUM_TILES = N // T
MAX_ITEMS = NUM_TILES + E

def _wid():
    return lax.axis_index("s") * NC + lax.axis_index("c")


@functools.cache
def _mesh():
    return plsc.VectorSubcoreMesh(core_axis_name="c", subcore_axis_name="s")


def _sc_params():
    return pltpu.CompilerParams(needs_layout_passes=False)


def _sc_count_body(assign_hbm, cnts_hbm, asg_v, cnt_v, lcnt_v):
    w = _wid()
    base = w * CPW
    pltpu.sync_copy(assign_hbm.at[pl.ds(base, CPW)], asg_v)
    lane = lax.iota(jnp.int32, LANES)
    ones = jnp.ones((LANES,), jnp.int32)
    for e in range(E):
        lcnt_v[e] = jnp.zeros((LANES,), jnp.int32)
    for j in range(CPW // LANES):
        vals = asg_v[pl.ds(j * LANES, LANES)]
        plsc.addupdate_scatter(lcnt_v, [vals, lane], ones)
    cnt = jnp.zeros((LANES,), jnp.int32)
    for e in range(E):
        cnt = cnt + jnp.where(lane == e, jnp.sum(lcnt_v[e]), 0)
    cnt_v[...] = cnt
    pltpu.sync_copy(cnt_v, cnts_hbm.at[w])


@functools.cache
def _sc_count():
    return pl.kernel(
        _sc_count_body,
        out_type=jax.ShapeDtypeStruct((NW, LANES), jnp.int32),
        mesh=_mesh(),
        scratch_types=[
            pltpu.VMEM((CPW,), jnp.int32),
            pltpu.VMEM((LANES,), jnp.int32),
            pltpu.VMEM((E, LANES), jnp.int32),
        ],
        compiler_params=_sc_params(),
    )


_NCHUNK = CPW // LANES


def _sc_dispatch_body(x_hbm, assign_hbm, cnts_hbm, xs_hbm, dest_hbm,
                      asg_v, cnt_v, dest_v, xbuf_v, lcnt_v,
                      ls0, ls1, ls2, ls3, ls4, ls5,
                      ss0, ss1, ss2, ss3, ss4, ss5):
    w = _wid()
    base = w * CPW
    lsem = (ls0, ls1, ls2, ls3, ls4, ls5)
    ssem = (ss0, ss1, ss2, ss3, ss4, ss5)

    h_ld = {}
    for j in range(3):
        h_ld[j] = pltpu.async_copy(
            x_hbm.at[pl.ds(base + j * LANES, LANES)], xbuf_v.at[j], lsem[j])

    pltpu.sync_copy(assign_hbm.at[pl.ds(base, CPW)], asg_v)
    pltpu.sync_copy(cnts_hbm, cnt_v)

    lane = lax.iota(jnp.int32, LANES)
    tot = jnp.zeros((LANES,), jnp.int32)
    pre = jnp.zeros((LANES,), jnp.int32)
    for wp in range(NW):
        row = cnt_v[wp]
        tot = tot + row
        pred = jnp.full((LANES,), wp, jnp.int32) < w
        pre = pre + jnp.where(pred, row, 0)
    off = plsc.cumsum(tot) - tot
    cur = off + pre

    ones = jnp.ones((LANES,), jnp.int32)
    for e in range(E):
        lcnt_v[e] = jnp.zeros((LANES,), jnp.int32)
    for j in range(_NCHUNK):
        vals = asg_v[pl.ds(j * LANES, LANES)]
        plsc.addupdate_scatter(lcnt_v, [vals, lane], ones)
    for e in range(E):
        row = lcnt_v[e]
        excl = plsc.cumsum(row) - row
        cur_e = jnp.sum(jnp.where(lane == e, cur, 0))
        lcnt_v[e] = excl + cur_e
    for j in range(_NCHUNK):
        vals = asg_v[pl.ds(j * LANES, LANES)]
        dest = plsc.load_gather(lcnt_v, [vals, lane])
        plsc.addupdate_scatter(lcnt_v, [vals, lane], ones)
        dest_v[j] = dest

    pltpu.sync_copy(dest_v, dest_hbm.at[w])

    h_sc = {}
    for j in range(_NCHUNK):
        bb = j % 6
        h_ld[j].wait()
        dv = dest_v[j]
        h_sc[j] = pltpu.async_copy(xbuf_v.at[bb], xs_hbm.at[dv], ssem[bb])
        if j + 3 < _NCHUNK:
            if j - 3 >= 0:
                h_sc[j - 3].wait()
            h_ld[j + 3] = pltpu.async_copy(
                x_hbm.at[pl.ds(base + (j + 3) * LANES, LANES)],
                xbuf_v.at[(j + 3) % 6], lsem[(j + 3) % 6])
    for j in range(_NCHUNK - 6, _NCHUNK):
        h_sc[j].wait()


@functools.cache
def _sc_dispatch():
    return pl.kernel(
        _sc_dispatch_body,
        out_type=(
            jax.ShapeDtypeStruct((N, D), jnp.float32),
            jax.ShapeDtypeStruct((NW, _NCHUNK, LANES), jnp.int32),
        ),
        mesh=_mesh(),
        scratch_types=[
            pltpu.VMEM((CPW,), jnp.int32),
            pltpu.VMEM((NW, LANES), jnp.int32),
            pltpu.VMEM((_NCHUNK, LANES), jnp.int32),
            pltpu.VMEM((6, LANES, D), jnp.float32),
            pltpu.VMEM((E, LANES), jnp.int32),
        ] + [pltpu.SemaphoreType.DMA] * 12,
        compiler_params=_sc_params(),
    )


def _sc_gather_body(ys_hbm, dest_hbm, out_hbm, dest_v, ybuf_v,
                    g0, g1, g2, g3, g4, g5, s0, s1, s2, s3, s4, s5):
    w = _wid()
    base = w * CPW
    gsem = (g0, g1, g2, g3, g4, g5)
    ssem = (s0, s1, s2, s3, s4, s5)
    pltpu.sync_copy(dest_hbm.at[w], dest_v)
    h_g, h_s = {}, {}
    for j in range(3):
        h_g[j] = pltpu.async_copy(
            ys_hbm.at[dest_v.at[j]], ybuf_v.at[j], gsem[j])
    for j in range(_NCHUNK):
        bb = j % 6
        h_g[j].wait()
        h_s[j] = pltpu.async_copy(
            ybuf_v.at[bb], out_hbm.at[pl.ds(base + j * LANES, LANES)],
            ssem[bb])
        if j + 3 < _NCHUNK:
            if j - 3 >= 0:
                h_s[j - 3].wait()
            h_g[j + 3] = pltpu.async_copy(
                ys_hbm.at[dest_v.at[j + 3]], ybuf_v.at[(j + 3) % 6],
                gsem[(j + 3) % 6])
    for j in range(_NCHUNK - 6, _NCHUNK):
        h_s[j].wait()


@functools.cache
def _sc_gather():
    return pl.kernel(
        _sc_gather_body,
        out_type=jax.ShapeDtypeStruct((N, D), jnp.float32),
        mesh=_mesh(),
        scratch_types=[
            pltpu.VMEM((_NCHUNK, LANES), jnp.int32),
            pltpu.VMEM((6, LANES, D), jnp.float32),
        ] + [pltpu.SemaphoreType.DMA] * 12,
        compiler_params=_sc_params(),
    )


def _gmm_body(it_tile, it_e, it_start, it_end, it_valid,
              x_ref, w_ref, b_ref, out_ref, wb_ref):
    i = pl.program_id(0)
    start = it_start[i]
    end = it_end[i]
    tile = it_tile[i]

    prev_e = it_e[jnp.maximum(i - 1, 0)]

    @pl.when((i == 0) | (it_e[i] != prev_e))
    def _():
        wb_ref[...] = w_ref[0].astype(jnp.bfloat16)

    full = (start <= tile * T) & (end >= (tile + 1) * T)

    @pl.when((it_valid[i] == 1) & full)
    def _():
        out_ref[...] = jnp.dot(x_ref[...].astype(jnp.bfloat16), wb_ref[...],
                               preferred_element_type=jnp.float32) + b_ref[0]

    @pl.when((it_valid[i] == 1) & jnp.logical_not(full))
    def _():
        rows = tile * T + lax.broadcasted_iota(jnp.int32, (T, 1), 0)
        m = (rows >= start) & (rows < end)
        y = jnp.dot(x_ref[...].astype(jnp.bfloat16), wb_ref[...],
                    preferred_element_type=jnp.float32) + b_ref[0]
        out_ref[...] = jnp.where(m, y, out_ref[...])


def _gmm(it_tile, it_e, it_start, it_end, it_valid, xs, W, b):
    grid_spec = pltpu.PrefetchScalarGridSpec(
        num_scalar_prefetch=5,
        grid=(MAX_ITEMS,),
        in_specs=[
            pl.BlockSpec((T, D), lambda i, tl, ex, st, en, va: (tl[i], 0)),
            pl.BlockSpec((1, D, D), lambda i, tl, ex, st, en, va: (ex[i], 0, 0)),
            pl.BlockSpec((1, 1, D), lambda i, tl, ex, st, en, va: (ex[i], 0, 0)),
        ],
        out_specs=pl.BlockSpec((T, D), lambda i, tl, ex, st, en, va: (tl[i], 0)),
        scratch_shapes=[pltpu.VMEM((D, D), jnp.bfloat16)],
    )
    return pl.pallas_call(
        _gmm_body,
        grid_spec=grid_spec,
        out_shape=jax.ShapeDtypeStruct((N, D), jnp.float32),
    )(it_tile, it_e, it_start, it_end, it_valid, xs, W,
      b.reshape(E, 1, D))


def kernel(x, W, b, assign):
    assign = assign.astype(jnp.int32)
    cnts = _sc_count()(assign)
    xs, dest = _sc_dispatch()(x, assign, cnts)

    totals = jnp.sum(cnts, axis=0)[:E]
    ends = jnp.cumsum(totals)
    starts = ends - totals
    first_tile = starts // T
    last_tile_ex = (ends + T - 1) // T
    ntiles = jnp.where(totals > 0, last_tile_ex - first_tile, 0)
    csum = jnp.cumsum(ntiles)
    total_items = csum[-1]
    i = jnp.arange(MAX_ITEMS, dtype=jnp.int32)
    ic = jnp.minimum(i, total_items - 1)
    e_of = jnp.sum((ic[:, None] >= csum[None, :]).astype(jnp.int32), axis=1)
    onehot = e_of[:, None] == jnp.arange(E, dtype=jnp.int32)[None, :]

    def pick(tbl):
        return jnp.sum(jnp.where(onehot, tbl[None, :], 0), axis=1).astype(jnp.int32)

    it_tile = pick(first_tile) + (ic - (pick(csum) - pick(ntiles)))
    it_e = e_of.astype(jnp.int32)
    it_start = pick(starts)
    it_end = pick(ends)
    it_valid = (i < total_items).astype(jnp.int32)

    ys = _gmm(it_tile, it_e, it_start, it_end, it_valid, xs, W, b)
    return _sc_gather()(ys, dest)
